# Initial kernel scaffold; baseline (speedup 1.0000x reference)
import functools

import jax
import jax.numpy as jnp
from jax import lax
from jax.experimental import pallas as pl
from jax.experimental.pallas import tpu as pltpu
from jax.experimental.pallas import tpu_sc as plsc

VOCAB = 1000000
D = 32
C = 4
B = 16384
L = 50

NC = 2
NS = 16
NW = NC * NS
BPW = B // NW
GROUP = 2 * L
NG = BPW * L // GROUP
NBUF = 2
LANES = 16

_INV_L = 1.0 / L


def _sc_bag_body(text_hbm, table_hbm, out_hbm, idx_v, rows_v, out_v, sem0, sem1):
    wid = lax.axis_index("s") * NC + lax.axis_index("c")
    sems = [sem0, sem1]

    pltpu.sync_copy(text_hbm.at[wid], idx_v)

    def _gather(j, slot):
        return pltpu.async_copy(
            table_hbm.at[idx_v.at[j]], rows_v.at[slot], sems[slot])

    for b in range(NBUF):
        _gather(b, b)

    def outer(i, _):
        for b in range(NBUF):
            j = i * NBUF + b
            pltpu.make_async_copy(
                table_hbm.at[idx_v.at[j]], rows_v.at[b], sems[b]).wait()
            for t in range(2):
                acc = [jnp.zeros((LANES,), jnp.float32) for _ in range(4)]
                for r in range(L):
                    row = t * L + r
                    acc[2 * (r % 2)] += rows_v[b, row, pl.ds(0, LANES)]
                    acc[2 * (r % 2) + 1] += rows_v[b, row, pl.ds(LANES, LANES)]
                bag = 2 * j + t
                out_v[bag, pl.ds(0, LANES)] = (acc[0] + acc[2]) * _INV_L
                out_v[bag, pl.ds(LANES, LANES)] = (acc[1] + acc[3]) * _INV_L

            @pl.when(j + NBUF < NG)
            def _():
                _gather(j + NBUF, b)
        return _

    lax.fori_loop(0, NG // NBUF, outer, None)

    pltpu.sync_copy(out_v, out_hbm.at[pl.ds(wid * BPW, BPW)])


_sc_bag = pl.kernel(
    _sc_bag_body,
    out_type=jax.ShapeDtypeStruct((B, D), jnp.float32),
    mesh=plsc.VectorSubcoreMesh(
        core_axis_name="c", subcore_axis_name="s", num_cores=NC,
        num_subcores=NS),
    scratch_types=[
        pltpu.VMEM((NG, GROUP), jnp.int32),
        pltpu.VMEM((NBUF, GROUP, D), jnp.float32),
        pltpu.VMEM((BPW, D), jnp.float32),
        pltpu.SemaphoreType.DMA,
        pltpu.SemaphoreType.DMA,
    ],
)


def _linear_body(x_ref, w_ref, b_ref, o_ref):
    o_ref[...] = (
        jnp.dot(x_ref[...], w_ref[...], preferred_element_type=jnp.float32)
        + b_ref[...])


_BM = 2048

_tc_linear = pl.pallas_call(
    _linear_body,
    grid=(B // _BM,),
    in_specs=[
        pl.BlockSpec((_BM, D), lambda i: (i, 0)),
        pl.BlockSpec((D, 128), lambda i: (0, 0)),
        pl.BlockSpec((1, 128), lambda i: (0, 0)),
    ],
    out_specs=pl.BlockSpec((_BM, 128), lambda i: (i, 0)),
    out_shape=jax.ShapeDtypeStruct((B, 128), jnp.float32),
)


def kernel(text, emb_weight, lin_w, lin_b):
    idx = text.astype(jnp.int32).reshape(NW, NG, GROUP)
    means = _sc_bag(idx, emb_weight)
    w_pad = jnp.zeros((D, 128), jnp.float32).at[:, :C].set(lin_w.T)
    b_pad = jnp.zeros((1, 128), jnp.float32).at[:, :C].set(lin_b)
    out = _tc_linear(means, w_pad, b_pad)
    return out[:, :C]


# speedup vs baseline: 2.5856x; 2.5856x over previous
import functools

import jax
import jax.numpy as jnp
from jax import lax
from jax.experimental import pallas as pl
from jax.experimental.pallas import tpu as pltpu
from jax.experimental.pallas import tpu_sc as plsc

VOCAB = 1000000
D = 32
C = 4
B = 16384
L = 50

NC = 2
NS = 16
NW = NC * NS
BPW = B // NW
GROUP = 2 * L
NG = BPW * L // GROUP
NBUF = 2
LANES = 16

_INV_L = 1.0 / L


def _sc_bag_body(text_hbm, table_hbm, out_hbm, idx_v, rows_v, out_v, sem0, sem1):
    wid = lax.axis_index("s") * NC + lax.axis_index("c")
    sems = [sem0, sem1]

    pltpu.sync_copy(text_hbm.at[wid], idx_v)

    def _gather(j, slot):
        return pltpu.async_copy(
            table_hbm.at[idx_v.at[j]], rows_v.at[slot], sems[slot])

    for b in range(NBUF):
        _gather(b, b)

    def outer(i, _):
        for b in range(NBUF):
            j = i * NBUF + b
            pltpu.make_async_copy(
                table_hbm.at[idx_v.at[j]], rows_v.at[b], sems[b]).wait()
            for t in range(2):
                acc = [jnp.zeros((LANES,), jnp.float32) for _ in range(4)]
                for r in range(L):
                    row = t * L + r
                    acc[2 * (r % 2)] += rows_v[b, row, pl.ds(0, LANES)]
                    acc[2 * (r % 2) + 1] += rows_v[b, row, pl.ds(LANES, LANES)]
                bag = 2 * j + t
                out_v[bag, pl.ds(0, LANES)] = (acc[0] + acc[2]) * _INV_L
                out_v[bag, pl.ds(LANES, LANES)] = (acc[1] + acc[3]) * _INV_L

            @pl.when(j + NBUF < NG)
            def _():
                _gather(j + NBUF, b)
        return _

    lax.fori_loop(0, NG // NBUF, outer, None)

    pltpu.sync_copy(out_v, out_hbm.at[pl.ds(wid * BPW, BPW)])


_sc_bag = pl.kernel(
    _sc_bag_body,
    out_type=jax.ShapeDtypeStruct((B, D), jnp.float32),
    mesh=plsc.VectorSubcoreMesh(
        core_axis_name="c", subcore_axis_name="s", num_cores=NC,
        num_subcores=NS),
    scratch_types=[
        pltpu.VMEM((NG, GROUP), jnp.int32),
        pltpu.VMEM((NBUF, GROUP, D), jnp.float32),
        pltpu.VMEM((BPW, D), jnp.float32),
        pltpu.SemaphoreType.DMA,
        pltpu.SemaphoreType.DMA,
    ],
    compiler_params=pltpu.CompilerParams(use_tc_tiling_on_sc=False),
)


def _linear_body(x_ref, w_ref, b_ref, o_ref):
    o_ref[...] = (
        jnp.dot(x_ref[...], w_ref[...], preferred_element_type=jnp.float32)
        + b_ref[...])


_BM = 2048

_tc_linear = pl.pallas_call(
    _linear_body,
    grid=(B // _BM,),
    in_specs=[
        pl.BlockSpec((_BM, D), lambda i: (i, 0)),
        pl.BlockSpec((D, 128), lambda i: (0, 0)),
        pl.BlockSpec((1, 128), lambda i: (0, 0)),
    ],
    out_specs=pl.BlockSpec((_BM, 128), lambda i: (i, 0)),
    out_shape=jax.ShapeDtypeStruct((B, 128), jnp.float32),
)


def kernel(text, emb_weight, lin_w, lin_b):
    idx = text.astype(jnp.int32).reshape(NW, NG, GROUP)
    means = _sc_bag(idx, emb_weight)
    w_pad = jnp.zeros((D, 128), jnp.float32).at[:, :C].set(lin_w.T)
    b_pad = jnp.zeros((1, 128), jnp.float32).at[:, :C].set(lin_b)
    out = _tc_linear(means, w_pad, b_pad)
    return out[:, :C]


# speedup vs baseline: 7.2821x; 2.8164x over previous
import jax
import jax.numpy as jnp
from jax import lax
from jax.experimental import pallas as pl
from jax.experimental.pallas import tpu as pltpu
from jax.experimental.pallas import tpu_sc as plsc

VOCAB = 1000000
D = 32
C = 4
B = 16384
L = 50

NC = 2
NS = 16
NW = NC * NS
BPW = B // NW
GROUP = 2 * L
NG = BPW * L // GROUP
_OFFS = (0, 16, 32, 48, 64, 80, 84)
NBUF = 8
LANES = 16

_INV_L = 1.0 / L


def _project_body(w_ref, b_ref, xt0_ref, xt1_ref, o_ref):
    p0 = jnp.dot(w_ref[...], xt0_ref[...],
                 preferred_element_type=jnp.float32) + b_ref[...]
    p1 = jnp.dot(w_ref[...], xt1_ref[...],
                 preferred_element_type=jnp.float32) + b_ref[...]
    o_ref[...] = jnp.concatenate([p0, p1], axis=0)


_PCOLS = 65536
VH = 8 * _PCOLS

_NHB = VH // _PCOLS

_tc_project = pl.pallas_call(
    _project_body,
    grid=(_NHB,),
    in_specs=[
        pl.BlockSpec((C, D), lambda i: (0, 0)),
        pl.BlockSpec((C, 1), lambda i: (0, 0)),
        pl.BlockSpec((D, _PCOLS), lambda i: (0, i)),
        pl.BlockSpec((D, _PCOLS), lambda i: (0, _NHB + i)),
    ],
    out_specs=pl.BlockSpec((2 * C, _PCOLS), lambda i: (0, i)),
    out_shape=jax.ShapeDtypeStruct((2 * C, VH), jnp.float32),
)


def _sc_bag_body(text_hbm, table_hbm, out_hbm, idx_v, idxc_v, vals_v, out_v,
                 sem0, sem1, sem2, sem3, sem4, sem5, sem6, sem7):
    wid = lax.axis_index("s") * NC + lax.axis_index("c")
    sems = [sem0, sem1, sem2, sem3, sem4, sem5, sem6, sem7]
    lane = jnp.arange(LANES, dtype=jnp.int32)

    pltpu.sync_copy(text_hbm.at[wid], idx_v)

    def _build_lists(j, slot):
        for o in _OFFS:
            raw = idx_v[j, pl.ds(o, LANES)]
            raw = raw + jnp.where(raw >= VH, jnp.int32(3 * VH), 0)
            for c in range(C):
                idxc_v[slot, pl.ds(c * GROUP + o, LANES)] = raw + (c * VH)

    def _fire(slot):
        pltpu.async_copy(
            table_hbm.at[idxc_v.at[slot]], vals_v.at[slot], sems[slot])

    def _drain(slot):
        pltpu.make_async_copy(
            table_hbm.at[idxc_v.at[slot]], vals_v.at[slot], sems[slot]).wait()

    for b in range(NBUF):
        _build_lists(b, b)
        _fire(b)

    mask_lt2 = lane < 2
    mask_ge2 = lane >= 2
    mask_ge12 = lane >= 12

    def outer(i, _):
        for b in range(NBUF):
            j = i * NBUF + b
            _drain(b)
            res = jnp.zeros((LANES,), jnp.float32)
            for c in range(C):
                for t in range(2):
                    if t == 0:
                        v0 = vals_v[b, pl.ds(c * GROUP + 0, LANES)]
                        v1 = vals_v[b, pl.ds(c * GROUP + 16, LANES)]
                        v2 = vals_v[b, pl.ds(c * GROUP + 32, LANES)]
                        v3 = jnp.where(mask_lt2,
                                       vals_v[b, pl.ds(c * GROUP + 48, LANES)], 0.0)
                    else:
                        v0 = jnp.where(mask_ge2,
                                       vals_v[b, pl.ds(c * GROUP + 48, LANES)], 0.0)
                        v1 = vals_v[b, pl.ds(c * GROUP + 64, LANES)]
                        v2 = vals_v[b, pl.ds(c * GROUP + 80, LANES)]
                        v3 = jnp.where(mask_ge12,
                                       vals_v[b, pl.ds(c * GROUP + 84, LANES)], 0.0)
                    acc = (v0 + v1) + (v2 + v3)
                    s = jnp.sum(acc)
                    res = jnp.where(lane == t * C + c, s * _INV_L, res)
            plsc.store_scatter(out_v, [8 * j + lane], res, mask=lane < 2 * C)

            @pl.when(j + NBUF < NG)
            def _():
                _build_lists(j + NBUF, b)
                _fire(b)
        return _

    lax.fori_loop(0, NG // NBUF, outer, None)

    pltpu.sync_copy(out_v, out_hbm.at[pl.ds(wid * BPW * C, BPW * C)])


_sc_bag = pl.kernel(
    _sc_bag_body,
    out_type=jax.ShapeDtypeStruct((B * C,), jnp.float32),
    mesh=plsc.VectorSubcoreMesh(
        core_axis_name="c", subcore_axis_name="s", num_cores=NC,
        num_subcores=NS),
    scratch_types=[
        pltpu.VMEM((NG, GROUP), jnp.int32),
        pltpu.VMEM((NBUF, C * GROUP), jnp.int32),
        pltpu.VMEM((NBUF, C * GROUP), jnp.float32),
        pltpu.VMEM((BPW * C,), jnp.float32),
        pltpu.SemaphoreType.DMA,
        pltpu.SemaphoreType.DMA,
        pltpu.SemaphoreType.DMA,
        pltpu.SemaphoreType.DMA,
        pltpu.SemaphoreType.DMA,
        pltpu.SemaphoreType.DMA,
        pltpu.SemaphoreType.DMA,
        pltpu.SemaphoreType.DMA,
    ],
    compiler_params=pltpu.CompilerParams(
        use_tc_tiling_on_sc=False, needs_layout_passes=False),
)


def kernel(text, emb_weight, lin_w, lin_b):
    idx = text.astype(jnp.int32).reshape(NW, NG, GROUP)
    emb_t = emb_weight.T
    pt = _tc_project(lin_w, lin_b.reshape(C, 1), emb_t, emb_t)
    p_flat = jnp.reshape(pt, (2 * C * VH,))
    out = _sc_bag(idx, p_flat)
    return jnp.reshape(out, (B, C))
